# baseline (speedup 1.0000x reference)
import functools

import jax
import jax.numpy as jnp
from jax import lax
from jax.experimental import pallas as pl
from jax.experimental.pallas import tpu as pltpu
from jax.experimental.pallas import tpu_sc as plsc

NUM_TASKS = 100000
NUM_EXPERTS = 128
BATCH = 16384

_NC = 2
_NS = 16
_NW = _NC * _NS
_B_PER_W = BATCH // _NW
_CHUNK = 128
_NCHUNK = _B_PER_W // _CHUNK
_L = 16


def _transpose_chunk(rows_c, trans_v, c, iotas):
    @plsc.parallel_loop(0, NUM_EXPERTS, unroll=8)
    def e_body(e):
        col = jnp.full((_L,), e, jnp.int32)
        for g in range(_CHUNK // _L):
            v = plsc.load_gather(rows_c, [iotas[g], col])
            trans_v[e, c, pl.ds(g * _L, _L)] = v


def _sc_gather_transpose(table, idx3):
    mesh = plsc.VectorSubcoreMesh(core_axis_name="c", subcore_axis_name="s")

    @functools.partial(
        pl.kernel,
        out_type=jax.ShapeDtypeStruct(
            (NUM_EXPERTS, BATCH // _CHUNK, _CHUNK), jnp.float32
        ),
        mesh=mesh,
        compiler_params=pltpu.CompilerParams(needs_layout_passes=False),
        scratch_types=[
            pltpu.VMEM((_NCHUNK, _CHUNK), jnp.int32),
            pltpu.VMEM((_CHUNK, NUM_EXPERTS), jnp.float32),
            pltpu.VMEM((_CHUNK, NUM_EXPERTS), jnp.float32),
            pltpu.VMEM((NUM_EXPERTS, _NCHUNK, _CHUNK), jnp.float32),
            pltpu.SemaphoreType.DMA,
            pltpu.SemaphoreType.DMA,
            pltpu.SemaphoreType.DMA,
        ],
    )
    def k(table_hbm, idx_hbm, out_hbm, idx_v, r0, r1, trans_v, g0, g1, wsem):
        rows = [r0, r1]
        gsems = [g0, g1]
        wid = lax.axis_index("s") * _NC + lax.axis_index("c")
        pltpu.sync_copy(idx_hbm.at[wid], idx_v)
        gathers = [
            pltpu.async_copy(table_hbm.at[idx_v.at[c]], rows[c], gsems[c])
            for c in range(2)
        ]
        iotas = [lax.iota(jnp.int32, _L) + g * _L for g in range(_CHUNK // _L)]
        for c in range(_NCHUNK):
            gathers[c].wait()
            _transpose_chunk(rows[c % 2], trans_v, c, iotas)
            if c + 2 < _NCHUNK:
                gathers.append(
                    pltpu.async_copy(
                        table_hbm.at[idx_v.at[c + 2]], rows[c % 2], gsems[c % 2]
                    )
                )
        pltpu.async_copy(
            trans_v, out_hbm.at[:, pl.ds(wid * _NCHUNK, _NCHUNK), :], wsem
        ).wait()

    return k(table, idx3)


def kernel(env_index, task_index_to_mask):
    idx = env_index.reshape(_NW, _NCHUNK, _CHUNK).astype(jnp.int32)
    out = _sc_gather_transpose(task_index_to_mask, idx)
    return out.reshape(NUM_EXPERTS, BATCH)[:, :, None]


# speedup vs baseline: 1.7156x; 1.7156x over previous
import functools

import jax
import jax.numpy as jnp
from jax import lax
from jax.experimental import pallas as pl
from jax.experimental.pallas import tpu as pltpu
from jax.experimental.pallas import tpu_sc as plsc

NUM_TASKS = 100000
NUM_EXPERTS = 128
BATCH = 16384

_NC = 2
_NS = 16
_NW = _NC * _NS
_B_PER_W = BATCH // _NW
_CHUNK = 128
_NCHUNK = _B_PER_W // _CHUNK
_L = 16


def _transpose_chunk(rows_c, trans_v, c, iota, rots):
    c_splat = jnp.full((_L,), c, jnp.int32)
    n_tiles = (NUM_EXPERTS // _L) * (_CHUNK // _L)

    @plsc.parallel_loop(0, n_tiles, unroll=2)
    def t_body(t):
        e0 = (t // (_CHUNK // _L)) * _L
        b0 = (t % (_CHUNK // _L)) * _L
        row = iota + b0
        for j in range(_L):
            col = rots[j] + e0
            v = plsc.load_gather(rows_c, [row, col])
            plsc.store_scatter(trans_v, [col, c_splat, row], v)


def _sc_gather_transpose(table, idx3):
    mesh = plsc.VectorSubcoreMesh(core_axis_name="c", subcore_axis_name="s")

    @functools.partial(
        pl.kernel,
        out_type=jax.ShapeDtypeStruct(
            (NUM_EXPERTS, BATCH // _CHUNK, _CHUNK), jnp.float32
        ),
        mesh=mesh,
        compiler_params=pltpu.CompilerParams(needs_layout_passes=False),
        scratch_types=[
            pltpu.VMEM((_NCHUNK, _CHUNK), jnp.int32),
            pltpu.VMEM((_CHUNK, NUM_EXPERTS), jnp.float32),
            pltpu.VMEM((_CHUNK, NUM_EXPERTS), jnp.float32),
            pltpu.VMEM((NUM_EXPERTS, _NCHUNK, _CHUNK), jnp.float32),
            pltpu.SemaphoreType.DMA,
            pltpu.SemaphoreType.DMA,
            pltpu.SemaphoreType.DMA,
        ],
    )
    def k(table_hbm, idx_hbm, out_hbm, idx_v, r0, r1, trans_v, g0, g1, wsem):
        rows = [r0, r1]
        gsems = [g0, g1]
        wid = lax.axis_index("s") * _NC + lax.axis_index("c")
        pltpu.sync_copy(idx_hbm.at[wid], idx_v)
        gathers = [
            pltpu.async_copy(table_hbm.at[idx_v.at[c]], rows[c], gsems[c])
            for c in range(2)
        ]
        iota = lax.iota(jnp.int32, _L)
        rots = [(iota + j) & (_L - 1) for j in range(_L)]
        for c in range(_NCHUNK):
            gathers[c].wait()
            _transpose_chunk(rows[c % 2], trans_v, c, iota, rots)
            if c + 2 < _NCHUNK:
                gathers.append(
                    pltpu.async_copy(
                        table_hbm.at[idx_v.at[c + 2]], rows[c % 2], gsems[c % 2]
                    )
                )
        pltpu.async_copy(
            trans_v, out_hbm.at[:, pl.ds(wid * _NCHUNK, _NCHUNK), :], wsem
        ).wait()

    return k(table, idx3)


def kernel(env_index, task_index_to_mask):
    idx = env_index.reshape(_NW, _NCHUNK, _CHUNK).astype(jnp.int32)
    out = _sc_gather_transpose(task_index_to_mask, idx)
    return out.reshape(NUM_EXPERTS, BATCH)[:, :, None]


# speedup vs baseline: 1.7174x; 1.0011x over previous
import functools

import jax
import jax.numpy as jnp
from jax import lax
from jax.experimental import pallas as pl
from jax.experimental.pallas import tpu as pltpu
from jax.experimental.pallas import tpu_sc as plsc

NUM_TASKS = 100000
NUM_EXPERTS = 128
BATCH = 16384

_NC = 2
_NS = 16
_NW = _NC * _NS
_B_PER_W = BATCH // _NW
_CHUNK = 128
_NCHUNK = _B_PER_W // _CHUNK
_L = 16


def _transpose_chunk(rows_c, trans_v, c, iota, rots):
    c_splat = jnp.full((_L,), c, jnp.int32)
    n_tiles = (NUM_EXPERTS // _L) * (_CHUNK // _L)

    @plsc.parallel_loop(0, n_tiles, unroll=2)
    def t_body(t):
        e0 = (t // (_CHUNK // _L)) * _L
        b0 = (t % (_CHUNK // _L)) * _L
        row = iota + b0
        for j in range(_L):
            col = rots[j] + e0
            v = plsc.load_gather(rows_c, [row, col])
            plsc.store_scatter(trans_v, [col, c_splat, row], v)


def _sc_gather_transpose(table, idx3):
    mesh = plsc.VectorSubcoreMesh(core_axis_name="c", subcore_axis_name="s")

    @functools.partial(
        pl.kernel,
        out_type=jax.ShapeDtypeStruct(
            (NUM_EXPERTS, BATCH // _CHUNK, _CHUNK), jnp.float32
        ),
        mesh=mesh,
        compiler_params=pltpu.CompilerParams(needs_layout_passes=False),
        scratch_types=[
            pltpu.VMEM((_NCHUNK, _CHUNK), jnp.int32),
            pltpu.VMEM((_CHUNK, NUM_EXPERTS), jnp.float32),
            pltpu.VMEM((_CHUNK, NUM_EXPERTS), jnp.float32),
            pltpu.VMEM((NUM_EXPERTS, _NCHUNK, _CHUNK), jnp.float32),
            pltpu.SemaphoreType.DMA,
            pltpu.SemaphoreType.DMA,
            pltpu.SemaphoreType.DMA,
        ],
    )
    def k(table_hbm, idx_hbm, out_hbm, idx_v, r0, r1, trans_v, g0, g1, wsem):
        rows = [r0, r1]
        gsems = [g0, g1]
        wid = lax.axis_index("s") * _NC + lax.axis_index("c")
        pltpu.sync_copy(idx_hbm.at[wid], idx_v)
        gathers = [
            pltpu.async_copy(table_hbm.at[idx_v.at[c]], rows[c], gsems[c])
            for c in range(2)
        ]
        iota = lax.iota(jnp.int32, _L)
        rots = [(iota + j) & (_L - 1) for j in range(_L)]
        half = _NCHUNK // 2
        w0 = None
        for c in range(_NCHUNK):
            gathers[c].wait()
            _transpose_chunk(rows[c % 2], trans_v, c, iota, rots)
            if c + 2 < _NCHUNK:
                gathers.append(
                    pltpu.async_copy(
                        table_hbm.at[idx_v.at[c + 2]], rows[c % 2], gsems[c % 2]
                    )
                )
            if c == half - 1:
                w0 = pltpu.async_copy(
                    trans_v.at[:, pl.ds(0, half), :],
                    out_hbm.at[:, pl.ds(wid * _NCHUNK, half), :],
                    wsem,
                )
        w1 = pltpu.async_copy(
            trans_v.at[:, pl.ds(half, half), :],
            out_hbm.at[:, pl.ds(wid * _NCHUNK + half, half), :],
            wsem,
        )
        w0.wait()
        w1.wait()

    return k(table, idx3)


def kernel(env_index, task_index_to_mask):
    idx = env_index.reshape(_NW, _NCHUNK, _CHUNK).astype(jnp.int32)
    out = _sc_gather_transpose(task_index_to_mask, idx)
    return out.reshape(NUM_EXPERTS, BATCH)[:, :, None]
